# baseline (speedup 1.0000x reference)
import numpy as np
import jax
import jax.numpy as jnp
from jax.experimental import pallas as pl
from jax.experimental.pallas import tpu as pltpu

_BT = 128


def _band_onehot(win, npos, off, pad):
    oh = np.zeros((3, win, npos), np.float32)
    for kx in range(3):
        for p in range(npos):
            jin = 2 * p + off + kx - pad
            if 0 <= jin < win:
                oh[kx, jin, p] = 1.0
    return oh


def _band_mats(wf, win, npos, pad, kpad, npad):
    cin, cout = wf.shape[2], wf.shape[3]
    halves = []
    for off in (0, 1):
        oh = jnp.asarray(_band_onehot(win, npos, off, pad))
        m = jnp.einsum("xjp,yxio->yjipo", oh, wf)
        m = m.reshape(3, win * cin, npos * cout)
        halves.append(jnp.pad(m, ((0, 0), (0, kpad - win * cin),
                                  (0, npad - npos * cout))))
    return jnp.concatenate(halves, axis=-1)


def _pool(s, nvalid, bt, half):
    s = s[0:nvalid].reshape(nvalid // 2, 2, bt, 2 * half)
    v = jnp.maximum(s[:, 0], s[:, 1])
    return jnp.maximum(v[..., 0:half], v[..., half:2 * half])


def _layer(xf, bt, rows, m_ref, nvalid, half):
    n = 2 * half
    y0 = jnp.dot(xf, m_ref[0], preferred_element_type=jnp.float32)
    y1 = jnp.dot(xf, m_ref[1], preferred_element_type=jnp.float32)
    y2 = jnp.dot(xf, m_ref[2], preferred_element_type=jnp.float32)
    y0 = y0.reshape(rows, bt, n)
    y1 = y1.reshape(rows, bt, n)
    y2 = y2.reshape(rows, bt, n)
    s = y0[0:nvalid] + y1[1:nvalid + 1] + y2[2:nvalid + 2]
    return _pool(s, nvalid, bt, half)


def _fused_kernel(x_ref, m1_ref, s1_ref, m2_ref, s2_ref, m3_ref, s3_ref,
                  wf_ref, bfc_ref, o_ref, x1_scr, x2_scr, x3_scr):
    bt = o_ref.shape[0]

    x1_scr[0:1] = jnp.zeros((1, bt, 28), jnp.float32)
    x1_scr[29:32] = jnp.zeros((3, bt, 28), jnp.float32)
    x1_scr[1:29] = x_ref[...]
    x1 = x1_scr[...]
    xw = jnp.concatenate([x1[0:30], x1[1:31], x1[2:32]], axis=2)
    y = jnp.dot(xw.reshape(30 * bt, 84), m1_ref[...],
                preferred_element_type=jnp.float32)
    p1 = _pool(y.reshape(30, bt, 512), 28, bt, 256)
    p1 = jnp.maximum(p1 + s1_ref[...], 0.0)

    x2_scr[0:1] = jnp.zeros((1, bt, 256), jnp.float32)
    x2_scr[15:16] = jnp.zeros((1, bt, 256), jnp.float32)
    x2_scr[1:15] = p1
    p2 = _layer(x2_scr[...].reshape(16 * bt, 256), bt, 16, m2_ref, 14, 256)
    p2 = jnp.maximum(p2 + s2_ref[...], 0.0)

    x3_scr[0:7] = p2
    x3_scr[7:8] = jnp.zeros((1, bt, 256), jnp.float32)
    p3 = _layer(x3_scr[...].reshape(8 * bt, 256), bt, 8, m3_ref, 4, 128)
    p3 = jnp.maximum(p3 + s3_ref[...], 0.0)

    xfc = jnp.concatenate([p3[0], p3[1]], axis=-1)
    o_ref[...] = bfc_ref[...] + jnp.dot(xfc, wf_ref[...],
                                        preferred_element_type=jnp.float32)


def kernel(x, w1, scale1, shift1, w2, scale2, shift2, w3, scale3, shift3,
           wfc, bfc):
    B = x.shape[0]
    bt = _BT if B >= _BT else B
    nt = -(-B // bt)
    bp = nt * bt
    xs = x.reshape(B, 28, 28).astype(jnp.float32)
    if bp != B:
        xs = jnp.pad(xs, ((0, bp - B), (0, 0), (0, 0)))
    xs = jnp.transpose(xs, (1, 0, 2))

    m1 = _band_mats(w1 * scale1.reshape(1, 1, 1, -1), 28, 14, 1, 28, 256)
    m1 = m1.reshape(3, 28, 2, 256).transpose(2, 0, 1, 3).reshape(2, 84, 256)
    m1 = jnp.concatenate([m1[0], m1[1]], axis=-1)
    m2 = _band_mats(w2 * scale2.reshape(1, 1, 1, -1), 14, 7, 1, 256, 256)
    m3 = _band_mats(w3 * scale3.reshape(1, 1, 1, -1), 7, 2, 0, 256, 128)
    sh1 = jnp.pad(jnp.tile(shift1.reshape(-1), 14), (0, 32)).reshape(1, 256)
    sh2 = jnp.pad(jnp.tile(shift2.reshape(-1), 7), (0, 32)).reshape(1, 256)
    sh3 = jnp.tile(shift3.reshape(-1), 2).reshape(1, 128)
    wfp = wfc.reshape(64, 2, 2, 62).transpose(1, 2, 0, 3).reshape(256, 62)

    out = pl.pallas_call(
        _fused_kernel,
        out_shape=jax.ShapeDtypeStruct((bp, 62), jnp.float32),
        grid=(nt,),
        in_specs=[
            pl.BlockSpec((28, bt, 28), lambda t: (0, t, 0)),
            pl.BlockSpec((84, 512), lambda t: (0, 0)),
            pl.BlockSpec((1, 256), lambda t: (0, 0)),
            pl.BlockSpec((3, 256, 512), lambda t: (0, 0, 0)),
            pl.BlockSpec((1, 256), lambda t: (0, 0)),
            pl.BlockSpec((3, 256, 256), lambda t: (0, 0, 0)),
            pl.BlockSpec((1, 128), lambda t: (0, 0)),
            pl.BlockSpec((256, 62), lambda t: (0, 0)),
            pl.BlockSpec((1, 62), lambda t: (0, 0)),
        ],
        out_specs=pl.BlockSpec((bt, 62), lambda t: (t, 0)),
        scratch_shapes=[
            pltpu.VMEM((32, bt, 28), jnp.float32),
            pltpu.VMEM((16, bt, 256), jnp.float32),
            pltpu.VMEM((8, bt, 256), jnp.float32),
        ],
        compiler_params=pltpu.CompilerParams(
            dimension_semantics=("parallel",)),
    )(xs, m1, sh1, m2, sh2, m3, sh3, wfp, bfc)
    return out[:B]


# speedup vs baseline: 1.0704x; 1.0704x over previous
import numpy as np
import jax
import jax.numpy as jnp
from jax.experimental import pallas as pl
from jax.experimental.pallas import tpu as pltpu

_BT = 256


def _band_onehot(win, npos, off, pad):
    oh = np.zeros((3, win, npos), np.float32)
    for kx in range(3):
        for p in range(npos):
            jin = 2 * p + off + kx - pad
            if 0 <= jin < win:
                oh[kx, jin, p] = 1.0
    return oh


def _band_mats(wf, win, npos, pad):
    cin, cout = wf.shape[2], wf.shape[3]
    mats = []
    for off in (0, 1):
        oh = jnp.asarray(_band_onehot(win, npos, off, pad))
        m = jnp.einsum("xjp,yxio->yjipo", oh, wf)
        mats.append(m.reshape(3, win * cin, npos * cout))
    return jnp.concatenate(mats, axis=0)


def _layer(xf, bt, rows, m_ref, nvalid):
    vs = []
    for par in range(2):
        y0 = jnp.dot(xf, m_ref[par * 3 + 0], preferred_element_type=jnp.float32)
        y1 = jnp.dot(xf, m_ref[par * 3 + 1], preferred_element_type=jnp.float32)
        y2 = jnp.dot(xf, m_ref[par * 3 + 2], preferred_element_type=jnp.float32)
        n = y0.shape[-1]
        y0 = y0.reshape(rows, bt, n)
        y1 = y1.reshape(rows, bt, n)
        y2 = y2.reshape(rows, bt, n)
        s = y0[0:nvalid] + y1[1:nvalid + 1] + y2[2:nvalid + 2]
        s = s.reshape(nvalid // 2, 2, bt, n)
        vs.append(jnp.maximum(s[:, 0], s[:, 1]))
    return jnp.maximum(vs[0], vs[1])


def _fused_kernel(x_ref, m1_ref, s1_ref, m2_ref, s2_ref, m3_ref, s3_ref,
                  wf_ref, bfc_ref, o_ref, x1_scr, x2_scr, x3_scr):
    bt = o_ref.shape[0]

    x1_scr[0:1] = jnp.zeros((1, bt, 28), jnp.float32)
    x1_scr[29:32] = jnp.zeros((3, bt, 28), jnp.float32)
    x1_scr[1:29] = x_ref[...]
    x1 = x1_scr[...]
    xw = jnp.concatenate([x1[0:30], x1[1:31], x1[2:32]], axis=2)
    xwf = xw.reshape(30 * bt, 84)
    vs1 = []
    for par in range(2):
        y = jnp.dot(xwf, m1_ref[par], preferred_element_type=jnp.float32)
        s = y.reshape(30, bt, 224)[0:28].reshape(14, 2, bt, 224)
        vs1.append(jnp.maximum(s[:, 0], s[:, 1]))
    p1 = jnp.maximum(vs1[0], vs1[1])
    p1 = jnp.maximum(p1 + s1_ref[...], 0.0)

    x2_scr[0:1] = jnp.zeros((1, bt, 224), jnp.float32)
    x2_scr[15:16] = jnp.zeros((1, bt, 224), jnp.float32)
    x2_scr[1:15] = p1
    p2 = _layer(x2_scr[...].reshape(16 * bt, 224), bt, 16, m2_ref, 14)
    p2 = jnp.maximum(p2 + s2_ref[...], 0.0)

    x3_scr[0:7] = p2.astype(jnp.float32)
    x3_scr[7:8] = jnp.zeros((1, bt, 224), jnp.float32)
    p3 = _layer(x3_scr[...].reshape(8 * bt, 224), bt, 8, m3_ref, 4)
    p3 = jnp.maximum(p3 + s3_ref[...], 0.0)

    logits = bfc_ref[...]
    p3b = p3.astype(jnp.float32)
    logits = logits + jnp.dot(p3[0], wf_ref[0],
                              preferred_element_type=jnp.float32)
    logits = logits + jnp.dot(p3[1], wf_ref[1],
                              preferred_element_type=jnp.float32)
    o_ref[...] = logits


def kernel(x, w1, scale1, shift1, w2, scale2, shift2, w3, scale3, shift3,
           wfc, bfc):
    B = x.shape[0]
    bt = _BT if B >= _BT else B
    nt = -(-B // bt)
    bp = nt * bt
    xs = x.reshape(B, 28, 28).astype(jnp.float32)
    if bp != B:
        xs = jnp.pad(xs, ((0, bp - B), (0, 0), (0, 0)))
    xs = jnp.transpose(xs, (1, 0, 2))

    m1 = _band_mats(w1 * scale1.reshape(1, 1, 1, -1), 28, 14, 1)
    m1 = m1.reshape(2, 84, 224)
    m2 = _band_mats(w2 * scale2.reshape(1, 1, 1, -1), 14, 7, 1)
    m3 = _band_mats(w3 * scale3.reshape(1, 1, 1, -1), 7, 2, 0)
    sh1 = jnp.tile(shift1.reshape(-1), 14).reshape(1, 224)
    sh2 = jnp.tile(shift2.reshape(-1), 7).reshape(1, 224)
    sh3 = jnp.tile(shift3.reshape(-1), 2).reshape(1, 128)
    wfp = wfc.reshape(64, 2, 2, 62).transpose(1, 2, 0, 3).reshape(2, 128, 62).astype(jnp.float32)

    out = pl.pallas_call(
        _fused_kernel,
        out_shape=jax.ShapeDtypeStruct((bp, 62), jnp.float32),
        grid=(nt,),
        in_specs=[
            pl.BlockSpec((28, bt, 28), lambda t: (0, t, 0)),
            pl.BlockSpec((2, 84, 224), lambda t: (0, 0, 0)),
            pl.BlockSpec((1, 224), lambda t: (0, 0)),
            pl.BlockSpec((6, 224, 224), lambda t: (0, 0, 0)),
            pl.BlockSpec((1, 224), lambda t: (0, 0)),
            pl.BlockSpec((6, 224, 128), lambda t: (0, 0, 0)),
            pl.BlockSpec((1, 128), lambda t: (0, 0)),
            pl.BlockSpec((2, 128, 62), lambda t: (0, 0, 0)),
            pl.BlockSpec((1, 62), lambda t: (0, 0)),
        ],
        out_specs=pl.BlockSpec((bt, 62), lambda t: (t, 0)),
        scratch_shapes=[
            pltpu.VMEM((32, bt, 28), jnp.float32),
            pltpu.VMEM((16, bt, 224), jnp.float32),
            pltpu.VMEM((8, bt, 224), jnp.float32),
        ],
        compiler_params=pltpu.CompilerParams(
            dimension_semantics=("parallel",)),
    )(xs, m1, sh1, m2, sh2, m3, sh3, wfp, bfc)
    return out[:B]


# speedup vs baseline: 1.0706x; 1.0001x over previous
import numpy as np
import jax
import jax.numpy as jnp
from jax.experimental import pallas as pl
from jax.experimental.pallas import tpu as pltpu

_BT = 512


def _band_onehot(win, npos, off, pad):
    oh = np.zeros((3, win, npos), np.float32)
    for kx in range(3):
        for p in range(npos):
            jin = 2 * p + off + kx - pad
            if 0 <= jin < win:
                oh[kx, jin, p] = 1.0
    return oh


def _band_mats(wf, win, npos, pad):
    cin, cout = wf.shape[2], wf.shape[3]
    mats = []
    for off in (0, 1):
        oh = jnp.asarray(_band_onehot(win, npos, off, pad))
        m = jnp.einsum("xjp,yxio->yjipo", oh, wf)
        mats.append(m.reshape(3, win * cin, npos * cout))
    return jnp.concatenate(mats, axis=0)


def _layer(xf, bt, rows, m_ref, nvalid):
    vs = []
    for par in range(2):
        y0 = jnp.dot(xf, m_ref[par * 3 + 0], preferred_element_type=jnp.float32)
        y1 = jnp.dot(xf, m_ref[par * 3 + 1], preferred_element_type=jnp.float32)
        y2 = jnp.dot(xf, m_ref[par * 3 + 2], preferred_element_type=jnp.float32)
        n = y0.shape[-1]
        y0 = y0.reshape(rows, bt, n)
        y1 = y1.reshape(rows, bt, n)
        y2 = y2.reshape(rows, bt, n)
        s = y0[0:nvalid] + y1[1:nvalid + 1] + y2[2:nvalid + 2]
        s = s.reshape(nvalid // 2, 2, bt, n)
        vs.append(jnp.maximum(s[:, 0], s[:, 1]))
    return jnp.maximum(vs[0], vs[1])


def _fused_kernel(x_ref, m1_ref, s1_ref, m2_ref, s2_ref, m3_ref, s3_ref,
                  wf_ref, bfc_ref, o_ref, x1_scr, x2_scr, x3_scr):
    bt = o_ref.shape[0]

    x1_scr[0:1] = jnp.zeros((1, bt, 28), jnp.float32)
    x1_scr[29:32] = jnp.zeros((3, bt, 28), jnp.float32)
    x1_scr[1:29] = x_ref[...]
    x1 = x1_scr[...]
    xw = jnp.concatenate([x1[0:30], x1[1:31], x1[2:32]], axis=2)
    xwf = xw.reshape(30 * bt, 84)
    vs1 = []
    for par in range(2):
        y = jnp.dot(xwf, m1_ref[par], preferred_element_type=jnp.float32)
        s = y.reshape(30, bt, 224)[0:28].reshape(14, 2, bt, 224)
        vs1.append(jnp.maximum(s[:, 0], s[:, 1]))
    p1 = jnp.maximum(vs1[0], vs1[1])
    p1 = jnp.maximum(p1 + s1_ref[...], 0.0)

    x2_scr[0:1] = jnp.zeros((1, bt, 224), jnp.float32)
    x2_scr[15:16] = jnp.zeros((1, bt, 224), jnp.float32)
    x2_scr[1:15] = p1
    p2 = _layer(x2_scr[...].reshape(16 * bt, 224), bt, 16, m2_ref, 14)
    p2 = jnp.maximum(p2 + s2_ref[...], 0.0)

    x3_scr[0:7] = p2
    x3_scr[7:8] = jnp.zeros((1, bt, 224), jnp.float32)
    p3 = _layer(x3_scr[...].reshape(8 * bt, 224), bt, 8, m3_ref, 4)
    p3 = jnp.maximum(p3 + s3_ref[...], 0.0)

    logits = bfc_ref[...]
    logits = logits + jnp.dot(p3[0], wf_ref[0],
                              preferred_element_type=jnp.float32)
    logits = logits + jnp.dot(p3[1], wf_ref[1],
                              preferred_element_type=jnp.float32)
    o_ref[...] = logits


def kernel(x, w1, scale1, shift1, w2, scale2, shift2, w3, scale3, shift3,
           wfc, bfc):
    B = x.shape[0]
    bt = _BT if B >= _BT else B
    nt = -(-B // bt)
    bp = nt * bt
    xs = x.reshape(B, 28, 28).astype(jnp.float32)
    if bp != B:
        xs = jnp.pad(xs, ((0, bp - B), (0, 0), (0, 0)))
    xs = jnp.transpose(xs, (1, 0, 2))

    m1 = _band_mats(w1 * scale1.reshape(1, 1, 1, -1), 28, 14, 1)
    m1 = m1.reshape(2, 84, 224)
    m2 = _band_mats(w2 * scale2.reshape(1, 1, 1, -1), 14, 7, 1)
    m3 = _band_mats(w3 * scale3.reshape(1, 1, 1, -1), 7, 2, 0)
    sh1 = jnp.tile(shift1.reshape(-1), 14).reshape(1, 224)
    sh2 = jnp.tile(shift2.reshape(-1), 7).reshape(1, 224)
    sh3 = jnp.tile(shift3.reshape(-1), 2).reshape(1, 128)
    wfp = wfc.reshape(64, 2, 2, 62).transpose(1, 2, 0, 3).reshape(2, 128, 62)

    out = pl.pallas_call(
        _fused_kernel,
        out_shape=jax.ShapeDtypeStruct((bp, 62), jnp.float32),
        grid=(nt,),
        in_specs=[
            pl.BlockSpec((28, bt, 28), lambda t: (0, t, 0)),
            pl.BlockSpec((2, 84, 224), lambda t: (0, 0, 0)),
            pl.BlockSpec((1, 224), lambda t: (0, 0)),
            pl.BlockSpec((6, 224, 224), lambda t: (0, 0, 0)),
            pl.BlockSpec((1, 224), lambda t: (0, 0)),
            pl.BlockSpec((6, 224, 128), lambda t: (0, 0, 0)),
            pl.BlockSpec((1, 128), lambda t: (0, 0)),
            pl.BlockSpec((2, 128, 62), lambda t: (0, 0, 0)),
            pl.BlockSpec((1, 62), lambda t: (0, 0)),
        ],
        out_specs=pl.BlockSpec((bt, 62), lambda t: (t, 0)),
        scratch_shapes=[
            pltpu.VMEM((32, bt, 28), jnp.float32),
            pltpu.VMEM((16, bt, 224), jnp.float32),
            pltpu.VMEM((8, bt, 224), jnp.float32),
        ],
        compiler_params=pltpu.CompilerParams(
            dimension_semantics=("parallel",)),
    )(xs, m1, sh1, m2, sh2, m3, sh3, wfp, bfc)
    return out[:B]
